# baseline (speedup 1.0000x reference)
import functools

import jax
import jax.numpy as jnp
from jax import lax
from jax.experimental import pallas as pl
from jax.experimental.pallas import tpu as pltpu
from jax.experimental.pallas import tpu_sc as plsc

N = 65536
D = 128
NT = 32
NCT = 64
L = 16
NC = 2
NS = 16
NW = NC * NS
NPW = N // NW
C = 64
NCHUNKS = NPW // C

_mesh = plsc.VectorSubcoreMesh(core_axis_name="c", subcore_axis_name="s")


@functools.partial(
    pl.kernel,
    mesh=_mesh,
    out_type=[
        jax.ShapeDtypeStruct((N * D,), jnp.float32),
        jax.ShapeDtypeStruct((3 * N * D,), jnp.float32),
    ],
    compiler_params=pltpu.CompilerParams(needs_layout_passes=False),
    scratch_types=[
        pltpu.VMEM((NT * D,), jnp.float32),
        pltpu.VMEM((NCT * D,), jnp.float32),
        pltpu.VMEM((NT * 3 * D,), jnp.float32),
        pltpu.VMEM((NPW,), jnp.int32),
        pltpu.VMEM((NPW,), jnp.int32),
        pltpu.VMEM((9 * NPW,), jnp.float32),
        pltpu.VMEM((C * D,), jnp.float32),
        pltpu.VMEM((C * D,), jnp.float32),
        pltpu.VMEM((3 * C * D,), jnp.float32),
        pltpu.VMEM((3 * C * D,), jnp.float32),
        pltpu.SemaphoreType.DMA,
        pltpu.SemaphoreType.DMA,
        pltpu.SemaphoreType.DMA,
    ],
)
def _sc_embed(nt_hbm, cid_hbm, rot_hbm, ts_hbm, cs_hbm, tvp_hbm,
              s_hbm, v_hbm,
              ts_v, cs_v, tvp_v, nt_v, cid_v, rot_v,
              s_buf0, s_buf1, v_buf0, v_buf1, sem_in, sem0, sem1):
    wid = lax.axis_index("s") * NC + lax.axis_index("c")
    base = wid * NPW

    copies = [
        pltpu.async_copy(ts_hbm, ts_v, sem_in),
        pltpu.async_copy(cs_hbm, cs_v, sem_in),
        pltpu.async_copy(tvp_hbm, tvp_v, sem_in),
        pltpu.async_copy(nt_hbm.at[pl.ds(base, NPW)], nt_v, sem_in),
        pltpu.async_copy(cid_hbm.at[pl.ds(base, NPW)], cid_v, sem_in),
    ] + [
        pltpu.async_copy(rot_hbm.at[pl.ds(k * N + base, NPW)],
                         rot_v.at[pl.ds(k * NPW, NPW)], sem_in)
        for k in range(9)
    ]
    for cp in copies:
        cp.wait()

    def do_chunk(g, h, s_buf, v_buf, sem):
        nbase = base + g * C

        @pl.when(h >= 1)
        def _():
            pltpu.make_async_copy(
                s_buf, s_hbm.at[pl.ds(nbase * D, C * D)], sem).wait()
            for i in range(3):
                pltpu.make_async_copy(
                    v_buf.at[pl.ds(i * (C * D), C * D)],
                    v_hbm.at[pl.ds(i * (N * D) + nbase * D, C * D)],
                    sem).wait()

        @plsc.parallel_loop(0, C // L, 1)
        def group_body(nb):
            gb = nb * L
            lb = g * C + gb
            nt16 = nt_v[pl.ds(lb, L)]
            cid16 = cid_v[pl.ds(lb, L)]
            rv = [rot_v[pl.ds(k * NPW + lb, L)] for k in range(9)]
            for m in range(L):
                ob = (gb + m) * D
                tsb = nt16[m] * D
                csb = cid16[m] * D
                for cb in range(D // L):
                    a = ts_v[pl.ds(tsb + cb * L, L)]
                    b = cs_v[pl.ds(csb + cb * L, L)]
                    s_buf[pl.ds(ob + cb * L, L)] = a + b
                r = [rv[k][m] for k in range(9)]
                tvb = nt16[m] * (3 * D)
                for db in range(D // L):
                    p = [tvp_v[pl.ds(tvb + j * D + db * L, L)]
                         for j in range(3)]
                    for i in range(3):
                        o = r[3 * i] * p[0] + r[3 * i + 1] * p[1] \
                            + r[3 * i + 2] * p[2]
                        v_buf[pl.ds(i * (C * D) + ob + db * L, L)] = o

        pltpu.async_copy(s_buf, s_hbm.at[pl.ds(nbase * D, C * D)], sem)
        for i in range(3):
            pltpu.async_copy(v_buf.at[pl.ds(i * (C * D), C * D)],
                             v_hbm.at[pl.ds(i * (N * D) + nbase * D, C * D)],
                             sem)

    def half_body(h, carry):
        do_chunk(2 * h, h, s_buf0, v_buf0, sem0)
        do_chunk(2 * h + 1, h, s_buf1, v_buf1, sem1)
        return carry

    lax.fori_loop(0, NCHUNKS // 2, half_body, 0)

    gl0, gl1 = NCHUNKS - 2, NCHUNKS - 1
    pltpu.make_async_copy(
        s_buf0, s_hbm.at[pl.ds((base + gl0 * C) * D, C * D)], sem0).wait()
    pltpu.make_async_copy(
        s_buf1, s_hbm.at[pl.ds((base + gl1 * C) * D, C * D)], sem1).wait()
    for i in range(3):
        pltpu.make_async_copy(
            v_buf0.at[pl.ds(i * (C * D), C * D)],
            v_hbm.at[pl.ds(i * (N * D) + (base + gl0 * C) * D, C * D)],
            sem0).wait()
        pltpu.make_async_copy(
            v_buf1.at[pl.ds(i * (C * D), C * D)],
            v_hbm.at[pl.ds(i * (N * D) + (base + gl1 * C) * D, C * D)],
            sem1).wait()


def kernel(node_type, rotmat, chain_id, type2scalar, type2vec, chain2scalar):
    nt = node_type.astype(jnp.int32)
    cid = chain_id.astype(jnp.int32)
    rot = rotmat.transpose(1, 2, 0).reshape(9 * N)
    ts = type2scalar.reshape(NT * D)
    cs = chain2scalar.reshape(NCT * D)
    tvp = type2vec.reshape(NT, D, 3).transpose(0, 2, 1).reshape(NT * 3 * D)
    s_flat, v_flat = _sc_embed(nt, cid, rot, ts, cs, tvp)
    return (s_flat.reshape(N, D),
            v_flat.reshape(3, N, D).transpose(1, 2, 0))


# speedup vs baseline: 1.1861x; 1.1861x over previous
import functools

import jax
import jax.numpy as jnp
from jax import lax
from jax.experimental import pallas as pl
from jax.experimental.pallas import tpu as pltpu
from jax.experimental.pallas import tpu_sc as plsc

N = 65536
D = 128
NT = 32
NCT = 64
L = 16
NC = 2
NS = 16
NW = NC * NS
NPW = N // NW
C = 32
NCHUNKS = NPW // C

_mesh = plsc.VectorSubcoreMesh(core_axis_name="c", subcore_axis_name="s")


@functools.partial(
    pl.kernel,
    mesh=_mesh,
    out_type=[
        jax.ShapeDtypeStruct((N, D), jnp.float32),
        jax.ShapeDtypeStruct((3, N, D), jnp.float32),
    ],
    compiler_params=pltpu.CompilerParams(needs_layout_passes=False),
    scratch_types=[
        pltpu.VMEM((NPW,), jnp.int32),
        pltpu.VMEM((NPW,), jnp.int32),
        pltpu.VMEM((9 * NPW,), jnp.float32),
        pltpu.VMEM((2, C, D), jnp.float32),
        pltpu.VMEM((2, C, D), jnp.float32),
        pltpu.VMEM((2, C, 3 * D), jnp.float32),
        pltpu.VMEM((2, C, D), jnp.float32),
        pltpu.VMEM((2, 3, C, D), jnp.float32),
        pltpu.SemaphoreType.DMA,
        pltpu.SemaphoreType.DMA,
        pltpu.SemaphoreType.DMA,
        pltpu.SemaphoreType.DMA,
        pltpu.SemaphoreType.DMA,
    ],
)
def _sc_embed(nt_hbm, cid_hbm, rot_hbm, ts_hbm, cs_hbm, tvp_hbm,
              s_hbm, v_hbm,
              nt_v, cid_v, rot_v,
              a_buf, b_buf, pv_buf, s_buf, v_buf,
              sem_in, gsem0, gsem1, osem0, osem1):
    wid = lax.axis_index("s") * NC + lax.axis_index("c")
    base = wid * NPW

    copies = [
        pltpu.async_copy(nt_hbm.at[pl.ds(base, NPW)], nt_v, sem_in),
        pltpu.async_copy(cid_hbm.at[pl.ds(base, NPW)], cid_v, sem_in),
    ] + [
        pltpu.async_copy(rot_hbm.at[pl.ds(k * N + base, NPW)],
                         rot_v.at[pl.ds(k * NPW, NPW)], sem_in)
        for k in range(9)
    ]
    for cp in copies:
        cp.wait()

    def issue_gathers(g, b, gsem):
        nt_idx = nt_v.at[pl.ds(g * C, C)]
        cid_idx = cid_v.at[pl.ds(g * C, C)]
        pltpu.async_copy(ts_hbm.at[nt_idx], a_buf.at[b], gsem)
        pltpu.async_copy(cs_hbm.at[cid_idx], b_buf.at[b], gsem)
        pltpu.async_copy(tvp_hbm.at[nt_idx], pv_buf.at[b], gsem)

    def wait_gathers(g, b, gsem):
        nt_idx = nt_v.at[pl.ds(g * C, C)]
        cid_idx = cid_v.at[pl.ds(g * C, C)]
        pltpu.make_async_copy(ts_hbm.at[nt_idx], a_buf.at[b], gsem).wait()
        pltpu.make_async_copy(cs_hbm.at[cid_idx], b_buf.at[b], gsem).wait()
        pltpu.make_async_copy(tvp_hbm.at[nt_idx], pv_buf.at[b], gsem).wait()

    def issue_outs(g, b, osem):
        nbase = base + g * C
        pltpu.async_copy(s_buf.at[b], s_hbm.at[pl.ds(nbase, C)], osem)
        for i in range(3):
            pltpu.async_copy(v_buf.at[b, i],
                             v_hbm.at[i, pl.ds(nbase, C)], osem)

    def wait_outs(g, b, osem):
        nbase = base + g * C
        pltpu.make_async_copy(
            s_buf.at[b], s_hbm.at[pl.ds(nbase, C)], osem).wait()
        for i in range(3):
            pltpu.make_async_copy(v_buf.at[b, i],
                                  v_hbm.at[i, pl.ds(nbase, C)], osem).wait()

    def compute(g, b):
        @plsc.parallel_loop(0, C // L, 1)
        def group_body(nb):
            gb = nb * L
            lb = g * C + gb
            rv = [rot_v[pl.ds(k * NPW + lb, L)] for k in range(9)]
            for m in range(L):
                row = gb + m
                for cb in range(D // L):
                    s_buf[b, row, pl.ds(cb * L, L)] = (
                        a_buf[b, row, pl.ds(cb * L, L)]
                        + b_buf[b, row, pl.ds(cb * L, L)])
                r = [rv[k][m] for k in range(9)]
                for db in range(D // L):
                    p = [pv_buf[b, row, pl.ds(j * D + db * L, L)]
                         for j in range(3)]
                    for i in range(3):
                        o = r[3 * i] * p[0] + r[3 * i + 1] * p[1] \
                            + r[3 * i + 2] * p[2]
                        v_buf[b, i, row, pl.ds(db * L, L)] = o

    issue_gathers(0, 0, gsem0)
    issue_gathers(1, 1, gsem1)

    def step(g, b, gsem, osem):
        wait_gathers(g, b, gsem)

        @pl.when(g >= 2)
        def _():
            wait_outs(g - 2, b, osem)

        compute(g, b)
        issue_outs(g, b, osem)

        @pl.when(g + 2 < NCHUNKS)
        def _():
            issue_gathers(g + 2, b, gsem)

    def half_body(h, carry):
        step(2 * h, 0, gsem0, osem0)
        step(2 * h + 1, 1, gsem1, osem1)
        return carry

    lax.fori_loop(0, NCHUNKS // 2, half_body, 0)

    wait_outs(NCHUNKS - 2, 0, osem0)
    wait_outs(NCHUNKS - 1, 1, osem1)


def kernel(node_type, rotmat, chain_id, type2scalar, type2vec, chain2scalar):
    nt = node_type.astype(jnp.int32)
    cid = chain_id.astype(jnp.int32)
    rot = rotmat.transpose(1, 2, 0).reshape(9 * N)
    ts = type2scalar
    cs = chain2scalar
    tvp = type2vec.reshape(NT, D, 3).transpose(0, 2, 1).reshape(NT, 3 * D)
    s_out, v_out = _sc_embed(nt, cid, rot, ts, cs, tvp)
    return s_out, v_out.transpose(1, 2, 0)
